# Initial kernel scaffold; baseline (speedup 1.0000x reference)
import jax
import jax.numpy as jnp
from jax.experimental import pallas as pl


def kernel(input_, weight):
    raise NotImplementedError("write your pallas kernel here")


# speedup vs baseline: 1.5071x; 1.5071x over previous
import functools

import jax
import jax.numpy as jnp
from jax import lax
from jax.experimental import pallas as pl
from jax.experimental.pallas import tpu as pltpu
from jax.experimental.pallas import tpu_sc as plsc

NC = 2
NS = 16
NW = NC * NS

B = 16384 * 20
D = 32
B_PER_W = B // NW
CHUNK = 2048
K = CHUNK // 128
N_CHUNKS = B_PER_W // CHUNK

_mesh = plsc.VectorSubcoreMesh(core_axis_name="c", subcore_axis_name="s")


@functools.partial(
    pl.kernel,
    out_type=jax.ShapeDtypeStruct((B, D), jnp.float32),
    mesh=_mesh,
    scratch_types=[
        pltpu.VMEM((K, 128), jnp.int32),
        pltpu.VMEM((CHUNK, D), jnp.float32),
        pltpu.SemaphoreType.DMA,
    ],
    compiler_params=pltpu.CompilerParams(use_tc_tiling_on_sc=False),
)
def _gather_kernel(idx_hbm, table_hbm, out_hbm, idx_v, rows_v, sem):
    wid = lax.axis_index("s") * NC + lax.axis_index("c")
    base = wid * B_PER_W

    def chunk_body(i, carry):
        off = base + i * CHUNK
        row0 = pl.multiple_of(off // 128, 8)
        pltpu.sync_copy(idx_hbm.at[pl.ds(row0, K)], idx_v)
        handles = []
        for j in range(K):
            handles.append(
                pltpu.async_copy(
                    table_hbm.at[idx_v.at[j]],
                    rows_v.at[pl.ds(j * 128, 128)],
                    sem,
                )
            )
        for h in handles:
            h.wait()
        pltpu.sync_copy(rows_v, out_hbm.at[pl.ds(off, CHUNK)])
        return carry

    lax.fori_loop(0, N_CHUNKS, chunk_body, 0)


def kernel(input_, weight):
    idx = jnp.reshape(input_.astype(jnp.int32), (B // 128, 128))
    out = _gather_kernel(idx, weight)
    return jnp.reshape(out, (*input_.shape, D))


# speedup vs baseline: 1.5148x; 1.0051x over previous
import functools

import jax
import jax.numpy as jnp
from jax import lax
from jax.experimental import pallas as pl
from jax.experimental.pallas import tpu as pltpu
from jax.experimental.pallas import tpu_sc as plsc

NC = 2
NS = 16
NW = NC * NS

B = 16384 * 20
D = 32
B_PER_W = B // NW
CHUNK = 1024
K = CHUNK // 128
N_CHUNKS = B_PER_W // CHUNK
IDX_ROWS = B_PER_W // 128

_mesh = plsc.VectorSubcoreMesh(core_axis_name="c", subcore_axis_name="s")


@functools.partial(
    pl.kernel,
    out_type=jax.ShapeDtypeStruct((B, D), jnp.float32),
    mesh=_mesh,
    scratch_types=[
        pltpu.VMEM((IDX_ROWS, 128), jnp.int32),
        pltpu.VMEM((CHUNK, D), jnp.float32),
        pltpu.VMEM((CHUNK, D), jnp.float32),
        pltpu.SemaphoreType.DMA,
        pltpu.SemaphoreType.DMA,
    ],
    compiler_params=pltpu.CompilerParams(use_tc_tiling_on_sc=False),
)
def _gather_kernel(idx_hbm, table_hbm, out_hbm, idx_v, rows0, rows1, sem0, sem1):
    wid = lax.axis_index("s") * NC + lax.axis_index("c")
    base = wid * B_PER_W

    idx_row0 = pl.multiple_of(base // 128, 8)
    pltpu.sync_copy(idx_hbm.at[pl.ds(idx_row0, IDX_ROWS)], idx_v)

    def fire(c, buf, sem):
        for j in range(K):
            pltpu.async_copy(
                table_hbm.at[idx_v.at[c * K + j]],
                buf.at[pl.ds(j * 128, 128)],
                sem,
            )

    def drain(buf, sem):
        pltpu.make_async_copy(out_hbm.at[pl.ds(0, CHUNK)], buf, sem).wait()

    def write(c, buf):
        off = pl.multiple_of(base + c * CHUNK, 8)
        pltpu.sync_copy(buf, out_hbm.at[pl.ds(off, CHUNK)])

    fire(0, rows0, sem0)

    def pair_body(p, carry):
        c0 = 2 * p
        c1 = c0 + 1
        c2 = c0 + 2
        fire(c1, rows1, sem1)
        drain(rows0, sem0)
        write(c0, rows0)

        @pl.when(c2 < N_CHUNKS)
        def _():
            fire(c2, rows0, sem0)

        drain(rows1, sem1)
        write(c1, rows1)
        return carry

    lax.fori_loop(0, N_CHUNKS // 2, pair_body, 0)


def kernel(input_, weight):
    idx = jnp.reshape(input_.astype(jnp.int32), (B // 128, 128))
    out = _gather_kernel(idx, weight)
    return jnp.reshape(out, (*input_.shape, D))
